# baseline (speedup 1.0000x reference)
import jax
import jax.numpy as jnp
from jax import lax
from jax.experimental import pallas as pl
from jax.experimental.pallas import tpu as pltpu
from jax.experimental.pallas import tpu_sc as plsc

N = 10000
D = 128
E = 320000
S = 4
NLAYER = 3

NP = 10240
NSC = 2
NT = 16
SNAP_PER_SC = S // NSC
EPT = E // NT
CH = 80
NCHUNK = EPT // CH
SCK = 10
SUPE = SCK * CH
NSUP = NCHUNK // SCK
RPT = NP // NT
BN = 1024
NB = NP // BN


def _get_mesh():
    return plsc.VectorSubcoreMesh(core_axis_name="c", subcore_axis_name="s",
                                  num_cores=NSC, num_subcores=NT)


def _zero16():
    return jnp.zeros((16,), jnp.float32)


def _sc_prep_body(src_hbm, dst_hbm, ew_hbm, selfnorm_hbm, norm_hbm,
                  srcadj_hbm,
                  srcb, dstb, ewb, dsmall, nbuf, abuf, dinv_t, sq_buf, zbuf,
                  deg_sh):
    c = lax.axis_index("c")
    t = lax.axis_index("s")

    def zz(i, carry):
        zbuf[pl.ds(i * 16, 16)] = _zero16()
        return carry
    lax.fori_loop(0, RPT // 16, zz, 0)

    for k in range(SNAP_PER_SC):
        s = c * SNAP_PER_SC + k
        base = s * E + t * EPT

        pltpu.sync_copy(src_hbm.at[pl.ds(base, EPT)], srcb)
        pltpu.sync_copy(dst_hbm.at[pl.ds(base, EPT)], dstb)
        pltpu.sync_copy(ew_hbm.at[pl.ds(base, EPT)], ewb)

        pltpu.sync_copy(zbuf, deg_sh.at[pl.ds(t * RPT, RPT)])
        plsc.subcore_barrier()

        def deg_step(i, carry):
            for g in range(CH // 16):
                dsmall[pl.ds(g * 16, 16)] = dstb[pl.ds(i * CH + g * 16, 16)]
            pltpu.sync_copy(ewb.at[pl.ds(i * CH, CH)],
                            deg_sh.at[dsmall], add=True)
            return carry
        lax.fori_loop(0, NCHUNK, deg_step, 0)
        plsc.subcore_barrier()

        pltpu.sync_copy(deg_sh, dinv_t)

        def rsq(i, carry):
            sl = pl.ds(i * 16, 16)
            dg = dinv_t[sl] + 1.0
            sq = 0.5 * (dg + 1.0)
            for _ in range(14):
                sq = 0.5 * (sq + dg / sq)
            dinv_t[sl] = 1.0 / sq
            return carry
        lax.fori_loop(0, NP // 16, rsq, 0)

        def sq(i, carry):
            v = dinv_t[pl.ds(t * RPT + i * 16, 16)]
            sq_buf[pl.ds(i * 16, 16)] = v * v
            return carry
        lax.fori_loop(0, RPT // 16, sq, 0)
        pltpu.sync_copy(sq_buf, selfnorm_hbm.at[pl.ds(s * NP + t * RPT, RPT)])

        sadd = s * NP

        def norm_step(u, carry):
            for j in range(SUPE // 16):
                sl = pl.ds(u * SUPE + j * 16, 16)
                osl = pl.ds(j * 16, 16)
                sv = srcb[sl]
                a = plsc.load_gather(dinv_t, [sv])
                b = plsc.load_gather(dinv_t, [dstb[sl]])
                nbuf[osl] = a * ewb[sl] * b
                abuf[osl] = sv + sadd
            off = base + u * SUPE
            pltpu.sync_copy(nbuf, norm_hbm.at[pl.ds(off, SUPE)])
            pltpu.sync_copy(abuf, srcadj_hbm.at[pl.ds(off, SUPE)])
            return carry
        lax.fori_loop(0, NSUP, norm_step, 0)
        plsc.subcore_barrier()


@jax.jit
def _sc_prep(src, dst, ew):
    return pl.kernel(
        _sc_prep_body,
        out_type=(
            jax.ShapeDtypeStruct((S * NP,), jnp.float32),
            jax.ShapeDtypeStruct((S * E,), jnp.float32),
            jax.ShapeDtypeStruct((S * E,), jnp.int32),
        ),
        mesh=_get_mesh(),
        scratch_types=[
            pltpu.VMEM((EPT,), jnp.int32),
            pltpu.VMEM((EPT,), jnp.int32),
            pltpu.VMEM((EPT,), jnp.float32),
            pltpu.VMEM((CH,), jnp.int32),
            pltpu.VMEM((SUPE,), jnp.float32),
            pltpu.VMEM((SUPE,), jnp.int32),
            pltpu.VMEM((NP,), jnp.float32),
            pltpu.VMEM((RPT,), jnp.float32),
            pltpu.VMEM((RPT,), jnp.float32),
            pltpu.VMEM_SHARED((NP,), jnp.float32),
        ],
        compiler_params=pltpu.CompilerParams(needs_layout_passes=False),
        name="sc_prep",
    )(src, dst, ew)


def _sc_spmm_body(hw_hbm, srcadj_hbm, dst_hbm, norm_hbm, out_hbm,
                  sadj, dstg, nbuf, dsm0, dsm1, rows0, rows1, srows0, srows1,
                  zrows, semg0, semg1, sems0, sems1, acc_sh):
    c = lax.axis_index("c")
    t = lax.axis_index("s")
    rows_refs = (rows0, rows1)
    srows_refs = (srows0, srows1)
    dsm_refs = (dsm0, dsm1)
    semg = (semg0, semg1)
    sems = (sems0, sems1)

    def zz(i, carry):
        for g in range(D // 16):
            zrows[i, pl.ds(g * 16, 16)] = _zero16()
        return carry
    lax.fori_loop(0, CH, zz, 0)

    for k in range(SNAP_PER_SC):
        s = c * SNAP_PER_SC + k
        for r in range(RPT // CH):
            pltpu.sync_copy(zrows, acc_sh.at[pl.ds(t * RPT + r * CH, CH)])
        plsc.subcore_barrier()

        base_e = s * E + t * EPT

        def super_body(u, carry):
            off = base_e + u * SUPE
            pltpu.sync_copy(srcadj_hbm.at[pl.ds(off, SUPE)], sadj)
            pltpu.sync_copy(norm_hbm.at[pl.ds(off, SUPE)], nbuf)
            pltpu.sync_copy(dst_hbm.at[pl.ds(off, SUPE)], dstg)
            scd = [None, None]
            gcur = pltpu.async_copy(
                hw_hbm.at[sadj.at[pl.ds(0, CH)]], rows0, semg0)
            for kk in range(SCK):
                b = kk & 1
                nb = 1 - b
                rb = rows_refs[b]
                sb = srows_refs[b]
                gcur.wait()
                if kk < SCK - 1:
                    if scd[nb] is not None:
                        scd[nb].wait()
                    gcur = pltpu.async_copy(
                        hw_hbm.at[sadj.at[pl.ds((kk + 1) * CH, CH)]],
                        rows_refs[nb], semg[nb])
                for g in range(CH // 16):
                    dsm_refs[b][pl.ds(g * 16, 16)] = (
                        dstg[pl.ds(kk * CH + g * 16, 16)])
                nbase = kk * CH

                def scale(j, carry2):
                    bc = plsc.load_gather(
                        nbuf, [jnp.zeros((16,), jnp.int32) + (nbase + j)])
                    for g in range(D // 32):
                        v16 = rb[j, pl.ds(g * 16, 16)]
                        v32 = plsc.bitcast(v16, jnp.bfloat16)
                        lo, hi = plsc.unpack(
                            v32, format=plsc.PackFormat.INTERLEAVED)
                        sb[j, pl.ds(g * 32, 16)] = lo * bc
                        sb[j, pl.ds(g * 32 + 16, 16)] = hi * bc
                    return carry2
                lax.fori_loop(0, CH, scale, 0)
                scd[b] = pltpu.async_copy(sb, acc_sh.at[dsm_refs[b]],
                                          sems[b], add=True)
            scd[0].wait()
            scd[1].wait()
            return carry
        lax.fori_loop(0, NSUP, super_body, 0)
        plsc.subcore_barrier()
        pltpu.sync_copy(acc_sh.at[pl.ds(t * RPT, RPT)],
                        out_hbm.at[pl.ds(s * NP + t * RPT, RPT)])
        plsc.subcore_barrier()


@jax.jit
def _sc_spmm(hw_flat, srcadj, dst, norm):
    return pl.kernel(
        _sc_spmm_body,
        out_type=jax.ShapeDtypeStruct((S * NP, D), jnp.float32),
        mesh=_get_mesh(),
        scratch_types=[
            pltpu.VMEM((SUPE,), jnp.int32),
            pltpu.VMEM((SUPE,), jnp.int32),
            pltpu.VMEM((SUPE,), jnp.float32),
            pltpu.VMEM((CH,), jnp.int32),
            pltpu.VMEM((CH,), jnp.int32),
            pltpu.VMEM((CH, D // 2), jnp.int32),
            pltpu.VMEM((CH, D // 2), jnp.int32),
            pltpu.VMEM((CH, D), jnp.float32),
            pltpu.VMEM((CH, D), jnp.float32),
            pltpu.VMEM((CH, D), jnp.float32),
            pltpu.SemaphoreType.DMA,
            pltpu.SemaphoreType.DMA,
            pltpu.SemaphoreType.DMA,
            pltpu.SemaphoreType.DMA,
            pltpu.VMEM_SHARED((NP, D), jnp.float32),
        ],
        compiler_params=pltpu.CompilerParams(needs_layout_passes=False,
                                             use_tc_tiling_on_sc=False),
        name="sc_spmm",
    )(hw_flat, srcadj, dst, norm)


def _tc_mm_body(x_ref, w_ref, o_ref):
    o_ref[...] = jnp.dot(x_ref[0], w_ref[...],
                         preferred_element_type=jnp.float32)[None]


@jax.jit
def _tc_matmul(h, w):
    return pl.pallas_call(
        _tc_mm_body,
        grid=(S, NB),
        in_specs=[
            pl.BlockSpec((1, BN, D), lambda sx, i: (sx, i, 0)),
            pl.BlockSpec((D, D), lambda sx, i: (0, 0)),
        ],
        out_specs=pl.BlockSpec((1, BN, D), lambda sx, i: (sx, i, 0)),
        out_shape=jax.ShapeDtypeStruct((S, NP, D), jnp.float32),
    )(h, w)


def _finish(acc, hw, sn, b, g, bb):
    tv = acc + sn * hw + b
    mu = jnp.mean(tv, axis=-1, keepdims=True)
    var = jnp.mean((tv - mu) ** 2, axis=-1, keepdims=True)
    hv = (tv - mu) / jnp.sqrt(var + 1e-5) * g + bb
    return jnp.maximum(hv, 0.0)


def _tc_mid_body(acc_ref, hw_ref, sn_ref, b_ref, g_ref, bb_ref, wn_ref, o_ref):
    hv = _finish(acc_ref[0], hw_ref[0], sn_ref[0], b_ref[...], g_ref[...],
                 bb_ref[...])
    o_ref[...] = jnp.dot(hv, wn_ref[...],
                         preferred_element_type=jnp.float32)[None]


def _tc_last_body(acc_ref, hw_ref, sn_ref, b_ref, g_ref, bb_ref, o_ref):
    o_ref[...] = _finish(acc_ref[0], hw_ref[0], sn_ref[0], b_ref[...],
                         g_ref[...], bb_ref[...])[None]


def _layer_specs(with_w):
    specs = [
        pl.BlockSpec((1, BN, D), lambda sx, i: (sx, i, 0)),
        pl.BlockSpec((1, BN, D), lambda sx, i: (sx, i, 0)),
        pl.BlockSpec((1, BN, 1), lambda sx, i: (sx, i, 0)),
        pl.BlockSpec((1, D), lambda sx, i: (0, 0)),
        pl.BlockSpec((1, D), lambda sx, i: (0, 0)),
        pl.BlockSpec((1, D), lambda sx, i: (0, 0)),
    ]
    if with_w:
        specs.append(pl.BlockSpec((D, D), lambda sx, i: (0, 0)))
    return specs


@jax.jit
def _tc_mid(acc, hw, sn, b, g, bb, wn):
    return pl.pallas_call(
        _tc_mid_body,
        grid=(S, NB),
        in_specs=_layer_specs(True),
        out_specs=pl.BlockSpec((1, BN, D), lambda sx, i: (sx, i, 0)),
        out_shape=jax.ShapeDtypeStruct((S, NP, D), jnp.float32),
    )(acc, hw, sn, b, g, bb, wn)


@jax.jit
def _tc_last(acc, hw, sn, b, g, bb):
    return pl.pallas_call(
        _tc_last_body,
        grid=(S, NB),
        in_specs=_layer_specs(False),
        out_specs=pl.BlockSpec((1, BN, D), lambda sx, i: (sx, i, 0)),
        out_shape=jax.ShapeDtypeStruct((S, NP, D), jnp.float32),
    )(acc, hw, sn, b, g, bb)


def _tc_gru_body(f_ref, wih_ref, whh_ref, bih_ref, bhh_ref, o_ref):
    h = jnp.zeros((BN, D), jnp.float32)
    wih = wih_ref[...]
    whh = whh_ref[...]
    for s in range(S):
        xt = f_ref[s]
        gi = jnp.dot(xt, wih, preferred_element_type=jnp.float32)
        gh = jnp.dot(h, whh, preferred_element_type=jnp.float32)
        r = jax.nn.sigmoid(gi[:, 0:D] + bih_ref[0] + gh[:, 0:D] + bhh_ref[0])
        z = jax.nn.sigmoid(gi[:, D:2 * D] + bih_ref[1]
                           + gh[:, D:2 * D] + bhh_ref[1])
        nv = jnp.tanh(gi[:, 2 * D:] + bih_ref[2]
                      + r * (gh[:, 2 * D:] + bhh_ref[2]))
        h = (1.0 - z) * nv + z * h
    o_ref[...] = h


@jax.jit
def _tc_gru(feats, wih_t, whh_t, bih, bhh):
    return pl.pallas_call(
        _tc_gru_body,
        grid=(NB,),
        in_specs=[
            pl.BlockSpec((S, BN, D), lambda i: (0, i, 0)),
            pl.BlockSpec((D, 3 * D), lambda i: (0, 0)),
            pl.BlockSpec((D, 3 * D), lambda i: (0, 0)),
            pl.BlockSpec((3, D), lambda i: (0, 0)),
            pl.BlockSpec((3, D), lambda i: (0, 0)),
        ],
        out_specs=pl.BlockSpec((BN, D), lambda i: (i, 0)),
        out_shape=jax.ShapeDtypeStruct((NP, D), jnp.float32),
    )(feats, wih_t, whh_t, bih, bhh)


def kernel(x, edge_index, edge_weight, gcn_W, gcn_b, ln_g, ln_b,
           W_ih, W_hh, b_ih, b_hh):
    src = edge_index[:, 0, :].reshape(S * E)
    dst = edge_index[:, 1, :].reshape(S * E)
    ew = edge_weight.reshape(S * E)
    xp = jnp.pad(x, ((0, 0), (0, NP - N), (0, 0)))

    selfnorm_flat, norm, srcadj = _sc_prep(src, dst, ew)
    selfnorm = selfnorm_flat.reshape(S, NP, 1)

    hw = _tc_matmul(xp, gcn_W[0])
    feats = None
    for l in range(NLAYER):
        hwbf = (hw.reshape(S, NP, D // 32, 2, 16)
                .transpose(0, 1, 2, 4, 3)
                .reshape(S * NP, D // 2, 2).astype(jnp.bfloat16))
        hwi = lax.bitcast_convert_type(hwbf, jnp.int32)
        acc = _sc_spmm(hwi, srcadj, dst, norm).reshape(S, NP, D)
        args = (acc, hw, selfnorm, gcn_b[l][None], ln_g[l][None],
                ln_b[l][None])
        if l < NLAYER - 1:
            hw = _tc_mid(*args, gcn_W[l + 1])
        else:
            feats = _tc_last(*args)

    hstate = _tc_gru(feats, W_ih.T, W_hh.T,
                     b_ih.reshape(3, D), b_hh.reshape(3, D))
    return hstate[:N]


# speedup vs baseline: 1.6767x; 1.6767x over previous
import jax
import jax.numpy as jnp
from jax import lax
from jax.experimental import pallas as pl
from jax.experimental.pallas import tpu as pltpu
from jax.experimental.pallas import tpu_sc as plsc

N = 10000
D = 128
E = 320000
S = 4
NLAYER = 3

NP = 10240
NSC = 2
NT = 16
SNAP_PER_SC = S // NSC
EPT = E // NT
CH = 80
NCHUNK = EPT // CH
SCK = 10
SUPE = SCK * CH
NSUP = NCHUNK // SCK
RPT = NP // NT
BN = 1024
NB = NP // BN


def _get_mesh():
    return plsc.VectorSubcoreMesh(core_axis_name="c", subcore_axis_name="s",
                                  num_cores=NSC, num_subcores=NT)


def _zero16():
    return jnp.zeros((16,), jnp.float32)


def _sc_prep_body(src_hbm, dst_hbm, ew_hbm, selfnorm_hbm, norm_hbm,
                  srcadj_hbm,
                  srcb, dstb, ewb, dsmall, nbuf, abuf, dinv_t, sq_buf, zbuf,
                  deg_sh):
    c = lax.axis_index("c")
    t = lax.axis_index("s")

    def zz(i, carry):
        zbuf[pl.ds(i * 16, 16)] = _zero16()
        return carry
    lax.fori_loop(0, RPT // 16, zz, 0)

    for k in range(SNAP_PER_SC):
        s = c * SNAP_PER_SC + k
        base = s * E + t * EPT

        pltpu.sync_copy(src_hbm.at[pl.ds(base, EPT)], srcb)
        pltpu.sync_copy(dst_hbm.at[pl.ds(base, EPT)], dstb)
        pltpu.sync_copy(ew_hbm.at[pl.ds(base, EPT)], ewb)

        pltpu.sync_copy(zbuf, deg_sh.at[pl.ds(t * RPT, RPT)])
        plsc.subcore_barrier()

        def deg_step(i, carry):
            for g in range(CH // 16):
                dsmall[pl.ds(g * 16, 16)] = dstb[pl.ds(i * CH + g * 16, 16)]
            pltpu.sync_copy(ewb.at[pl.ds(i * CH, CH)],
                            deg_sh.at[dsmall], add=True)
            return carry
        lax.fori_loop(0, NCHUNK, deg_step, 0)
        plsc.subcore_barrier()

        pltpu.sync_copy(deg_sh, dinv_t)

        def rsq(i, carry):
            sl = pl.ds(i * 16, 16)
            dg = dinv_t[sl] + 1.0
            sq = 0.5 * (dg + 1.0)
            for _ in range(14):
                sq = 0.5 * (sq + dg / sq)
            dinv_t[sl] = 1.0 / sq
            return carry
        lax.fori_loop(0, NP // 16, rsq, 0)

        def sq(i, carry):
            v = dinv_t[pl.ds(t * RPT + i * 16, 16)]
            sq_buf[pl.ds(i * 16, 16)] = v * v
            return carry
        lax.fori_loop(0, RPT // 16, sq, 0)
        pltpu.sync_copy(sq_buf, selfnorm_hbm.at[pl.ds(s * NP + t * RPT, RPT)])

        sadd = s * NP

        def norm_step(u, carry):
            for j in range(SUPE // 16):
                sl = pl.ds(u * SUPE + j * 16, 16)
                osl = pl.ds(j * 16, 16)
                sv = srcb[sl]
                a = plsc.load_gather(dinv_t, [sv])
                b = plsc.load_gather(dinv_t, [dstb[sl]])
                nbuf[osl] = a * ewb[sl] * b
                abuf[osl] = sv + sadd
            off = base + u * SUPE
            pltpu.sync_copy(nbuf, norm_hbm.at[pl.ds(off, SUPE)])
            pltpu.sync_copy(abuf, srcadj_hbm.at[pl.ds(off, SUPE)])
            return carry
        lax.fori_loop(0, NSUP, norm_step, 0)
        plsc.subcore_barrier()


@jax.jit
def _sc_prep(src, dst, ew):
    return pl.kernel(
        _sc_prep_body,
        out_type=(
            jax.ShapeDtypeStruct((S * NP,), jnp.float32),
            jax.ShapeDtypeStruct((S * E,), jnp.float32),
            jax.ShapeDtypeStruct((S * E,), jnp.int32),
        ),
        mesh=_get_mesh(),
        scratch_types=[
            pltpu.VMEM((EPT,), jnp.int32),
            pltpu.VMEM((EPT,), jnp.int32),
            pltpu.VMEM((EPT,), jnp.float32),
            pltpu.VMEM((CH,), jnp.int32),
            pltpu.VMEM((SUPE,), jnp.float32),
            pltpu.VMEM((SUPE,), jnp.int32),
            pltpu.VMEM((NP,), jnp.float32),
            pltpu.VMEM((RPT,), jnp.float32),
            pltpu.VMEM((RPT,), jnp.float32),
            pltpu.VMEM_SHARED((NP,), jnp.float32),
        ],
        compiler_params=pltpu.CompilerParams(needs_layout_passes=False),
        name="sc_prep",
    )(src, dst, ew)


def _sc_spmm_body(hw_hbm, srcadj_hbm, dst_hbm, norm_hbm, out_hbm,
                  sadj, dstg, nbuf, dsm0, dsm1, rows0, rows1,
                  zrows, semg0, semg1, sems0, sems1, acc_sh):
    c = lax.axis_index("c")
    t = lax.axis_index("s")
    rows_refs = (rows0, rows1)
    dsm_refs = (dsm0, dsm1)
    semg = (semg0, semg1)
    sems = (sems0, sems1)

    def zz(i, carry):
        for g in range(D // 16):
            zrows[i, pl.ds(g * 16, 16)] = _zero16()
        return carry
    lax.fori_loop(0, CH, zz, 0)

    for k in range(SNAP_PER_SC):
        s = c * SNAP_PER_SC + k
        for r in range(RPT // CH):
            pltpu.sync_copy(zrows, acc_sh.at[pl.ds(t * RPT + r * CH, CH)])
        plsc.subcore_barrier()

        base_e = s * E + t * EPT

        def super_body(u, carry):
            off = base_e + u * SUPE
            pltpu.sync_copy(srcadj_hbm.at[pl.ds(off, SUPE)], sadj)
            pltpu.sync_copy(norm_hbm.at[pl.ds(off, SUPE)], nbuf)
            pltpu.sync_copy(dst_hbm.at[pl.ds(off, SUPE)], dstg)
            scd = [None, None]
            gcur = pltpu.async_copy(
                hw_hbm.at[sadj.at[pl.ds(0, CH)]], rows0, semg0)
            for kk in range(SCK):
                b = kk & 1
                nb = 1 - b
                rb = rows_refs[b]
                gcur.wait()
                if kk < SCK - 1:
                    if scd[nb] is not None:
                        scd[nb].wait()
                    gcur = pltpu.async_copy(
                        hw_hbm.at[sadj.at[pl.ds((kk + 1) * CH, CH)]],
                        rows_refs[nb], semg[nb])
                for g in range(CH // 16):
                    dsm_refs[b][pl.ds(g * 16, 16)] = (
                        dstg[pl.ds(kk * CH + g * 16, 16)])
                nbase = kk * CH

                def scale(jj, carry2):
                    for e in range(2):
                        j = jj * 2 + e
                        bc = plsc.load_gather(
                            nbuf,
                            [jnp.zeros((16,), jnp.int32) + (nbase + j)])
                        vals = [rb[j, pl.ds(g * 16, 16)]
                                for g in range(D // 16)]
                        for g in range(D // 16):
                            rb[j, pl.ds(g * 16, 16)] = vals[g] * bc
                    return carry2
                lax.fori_loop(0, CH // 2, scale, 0)
                scd[b] = pltpu.async_copy(rb, acc_sh.at[dsm_refs[b]],
                                          sems[b], add=True)
            scd[0].wait()
            scd[1].wait()
            return carry
        lax.fori_loop(0, NSUP, super_body, 0)
        plsc.subcore_barrier()
        pltpu.sync_copy(acc_sh.at[pl.ds(t * RPT, RPT)],
                        out_hbm.at[pl.ds(s * NP + t * RPT, RPT)])
        plsc.subcore_barrier()


@jax.jit
def _sc_spmm(hw_flat, srcadj, dst, norm):
    return pl.kernel(
        _sc_spmm_body,
        out_type=jax.ShapeDtypeStruct((S * NP, D), jnp.float32),
        mesh=_get_mesh(),
        scratch_types=[
            pltpu.VMEM((SUPE,), jnp.int32),
            pltpu.VMEM((SUPE,), jnp.int32),
            pltpu.VMEM((SUPE,), jnp.float32),
            pltpu.VMEM((CH,), jnp.int32),
            pltpu.VMEM((CH,), jnp.int32),
            pltpu.VMEM((CH, D), jnp.float32),
            pltpu.VMEM((CH, D), jnp.float32),
            pltpu.VMEM((CH, D), jnp.float32),
            pltpu.SemaphoreType.DMA,
            pltpu.SemaphoreType.DMA,
            pltpu.SemaphoreType.DMA,
            pltpu.SemaphoreType.DMA,
            pltpu.VMEM_SHARED((NP, D), jnp.float32),
        ],
        compiler_params=pltpu.CompilerParams(needs_layout_passes=False),
        name="sc_spmm",
    )(hw_flat, srcadj, dst, norm)


def _tc_mm_body(x_ref, w_ref, o_ref):
    o_ref[...] = jnp.dot(x_ref[0], w_ref[...],
                         preferred_element_type=jnp.float32)[None]


@jax.jit
def _tc_matmul(h, w):
    return pl.pallas_call(
        _tc_mm_body,
        grid=(S, NB),
        in_specs=[
            pl.BlockSpec((1, BN, D), lambda sx, i: (sx, i, 0)),
            pl.BlockSpec((D, D), lambda sx, i: (0, 0)),
        ],
        out_specs=pl.BlockSpec((1, BN, D), lambda sx, i: (sx, i, 0)),
        out_shape=jax.ShapeDtypeStruct((S, NP, D), jnp.float32),
    )(h, w)


def _finish(acc, hw, sn, b, g, bb):
    tv = acc + sn * hw + b
    mu = jnp.mean(tv, axis=-1, keepdims=True)
    var = jnp.mean((tv - mu) ** 2, axis=-1, keepdims=True)
    hv = (tv - mu) / jnp.sqrt(var + 1e-5) * g + bb
    return jnp.maximum(hv, 0.0)


def _tc_mid_body(acc_ref, hw_ref, sn_ref, b_ref, g_ref, bb_ref, wn_ref, o_ref):
    hv = _finish(acc_ref[0], hw_ref[0], sn_ref[0], b_ref[...], g_ref[...],
                 bb_ref[...])
    o_ref[...] = jnp.dot(hv, wn_ref[...],
                         preferred_element_type=jnp.float32)[None]


def _tc_last_body(acc_ref, hw_ref, sn_ref, b_ref, g_ref, bb_ref, o_ref):
    o_ref[...] = _finish(acc_ref[0], hw_ref[0], sn_ref[0], b_ref[...],
                         g_ref[...], bb_ref[...])[None]


def _layer_specs(with_w):
    specs = [
        pl.BlockSpec((1, BN, D), lambda sx, i: (sx, i, 0)),
        pl.BlockSpec((1, BN, D), lambda sx, i: (sx, i, 0)),
        pl.BlockSpec((1, BN, 1), lambda sx, i: (sx, i, 0)),
        pl.BlockSpec((1, D), lambda sx, i: (0, 0)),
        pl.BlockSpec((1, D), lambda sx, i: (0, 0)),
        pl.BlockSpec((1, D), lambda sx, i: (0, 0)),
    ]
    if with_w:
        specs.append(pl.BlockSpec((D, D), lambda sx, i: (0, 0)))
    return specs


@jax.jit
def _tc_mid(acc, hw, sn, b, g, bb, wn):
    return pl.pallas_call(
        _tc_mid_body,
        grid=(S, NB),
        in_specs=_layer_specs(True),
        out_specs=pl.BlockSpec((1, BN, D), lambda sx, i: (sx, i, 0)),
        out_shape=jax.ShapeDtypeStruct((S, NP, D), jnp.float32),
    )(acc, hw, sn, b, g, bb, wn)


@jax.jit
def _tc_last(acc, hw, sn, b, g, bb):
    return pl.pallas_call(
        _tc_last_body,
        grid=(S, NB),
        in_specs=_layer_specs(False),
        out_specs=pl.BlockSpec((1, BN, D), lambda sx, i: (sx, i, 0)),
        out_shape=jax.ShapeDtypeStruct((S, NP, D), jnp.float32),
    )(acc, hw, sn, b, g, bb)


def _tc_gru_body(f_ref, wih_ref, whh_ref, bih_ref, bhh_ref, o_ref):
    h = jnp.zeros((BN, D), jnp.float32)
    wih = wih_ref[...]
    whh = whh_ref[...]
    for s in range(S):
        xt = f_ref[s]
        gi = jnp.dot(xt, wih, preferred_element_type=jnp.float32)
        gh = jnp.dot(h, whh, preferred_element_type=jnp.float32)
        r = jax.nn.sigmoid(gi[:, 0:D] + bih_ref[0] + gh[:, 0:D] + bhh_ref[0])
        z = jax.nn.sigmoid(gi[:, D:2 * D] + bih_ref[1]
                           + gh[:, D:2 * D] + bhh_ref[1])
        nv = jnp.tanh(gi[:, 2 * D:] + bih_ref[2]
                      + r * (gh[:, 2 * D:] + bhh_ref[2]))
        h = (1.0 - z) * nv + z * h
    o_ref[...] = h


@jax.jit
def _tc_gru(feats, wih_t, whh_t, bih, bhh):
    return pl.pallas_call(
        _tc_gru_body,
        grid=(NB,),
        in_specs=[
            pl.BlockSpec((S, BN, D), lambda i: (0, i, 0)),
            pl.BlockSpec((D, 3 * D), lambda i: (0, 0)),
            pl.BlockSpec((D, 3 * D), lambda i: (0, 0)),
            pl.BlockSpec((3, D), lambda i: (0, 0)),
            pl.BlockSpec((3, D), lambda i: (0, 0)),
        ],
        out_specs=pl.BlockSpec((BN, D), lambda i: (i, 0)),
        out_shape=jax.ShapeDtypeStruct((NP, D), jnp.float32),
    )(feats, wih_t, whh_t, bih, bhh)


def kernel(x, edge_index, edge_weight, gcn_W, gcn_b, ln_g, ln_b,
           W_ih, W_hh, b_ih, b_hh):
    src = edge_index[:, 0, :].reshape(S * E)
    dst = edge_index[:, 1, :].reshape(S * E)
    ew = edge_weight.reshape(S * E)
    xp = jnp.pad(x, ((0, 0), (0, NP - N), (0, 0)))

    selfnorm_flat, norm, srcadj = _sc_prep(src, dst, ew)
    selfnorm = selfnorm_flat.reshape(S, NP, 1)

    hw = _tc_matmul(xp, gcn_W[0])
    feats = None
    for l in range(NLAYER):
        acc = _sc_spmm(hw.reshape(S * NP, D), srcadj, dst,
                       norm).reshape(S, NP, D)
        args = (acc, hw, selfnorm, gcn_b[l][None], ln_g[l][None],
                ln_b[l][None])
        if l < NLAYER - 1:
            hw = _tc_mid(*args, gcn_W[l + 1])
        else:
            feats = _tc_last(*args)

    hstate = _tc_gru(feats, W_ih.T, W_hh.T,
                     b_ih.reshape(3, D), b_hh.reshape(3, D))
    return hstate[:N]


# speedup vs baseline: 1.7546x; 1.0465x over previous
import jax
import jax.numpy as jnp
from jax import lax
from jax.experimental import pallas as pl
from jax.experimental.pallas import tpu as pltpu
from jax.experimental.pallas import tpu_sc as plsc

N = 10000
D = 128
E = 320000
S = 4
NLAYER = 3

NP = 10240
NSC = 2
NT = 16
SNAP_PER_SC = S // NSC
EPT = E // NT
CH = 80
NCHUNK = EPT // CH
SCK = 10
SUPE = SCK * CH
NSUP = NCHUNK // SCK
RPT = NP // NT
BN = 1024
NB = NP // BN


def _get_mesh():
    return plsc.VectorSubcoreMesh(core_axis_name="c", subcore_axis_name="s",
                                  num_cores=NSC, num_subcores=NT)


def _zero16():
    return jnp.zeros((16,), jnp.float32)


def _sc_prep_body(src_hbm, dst_hbm, ew_hbm, selfnorm_hbm, norm_hbm,
                  srcadj_hbm,
                  srcb, dstb, ewb, dsmall, dsmall2, nbuf, abuf, dinv_t,
                  sq_buf, zbuf, semd0, semd1, deg_sh):
    c = lax.axis_index("c")
    t = lax.axis_index("s")

    def zz(i, carry):
        zbuf[pl.ds(i * 16, 16)] = _zero16()
        return carry
    lax.fori_loop(0, RPT // 16, zz, 0)

    for k in range(SNAP_PER_SC):
        s = c * SNAP_PER_SC + k
        base = s * E + t * EPT

        pltpu.sync_copy(src_hbm.at[pl.ds(base, EPT)], srcb)
        pltpu.sync_copy(dst_hbm.at[pl.ds(base, EPT)], dstb)
        pltpu.sync_copy(ew_hbm.at[pl.ds(base, EPT)], ewb)

        pltpu.sync_copy(zbuf, deg_sh.at[pl.ds(t * RPT, RPT)])
        plsc.subcore_barrier()

        def deg_step(i, carry):
            for e in range(2):
                kk = i * 2 + e
                dsm = dsmall if e == 0 else dsmall2
                sem = semd0 if e == 0 else semd1

                @pl.when(i >= 1)
                def _():
                    pltpu.make_async_copy(ewb.at[pl.ds(0, CH)],
                                          deg_sh.at[dsm], sem).wait()
                for g in range(CH // 16):
                    dsm[pl.ds(g * 16, 16)] = dstb[pl.ds(kk * CH + g * 16, 16)]
                pltpu.async_copy(ewb.at[pl.ds(kk * CH, CH)],
                                 deg_sh.at[dsm], sem, add=True)
            return carry
        lax.fori_loop(0, NCHUNK // 2, deg_step, 0)
        pltpu.make_async_copy(ewb.at[pl.ds(0, CH)],
                              deg_sh.at[dsmall], semd0).wait()
        pltpu.make_async_copy(ewb.at[pl.ds(0, CH)],
                              deg_sh.at[dsmall2], semd1).wait()
        plsc.subcore_barrier()

        pltpu.sync_copy(deg_sh, dinv_t)

        def rsq(i, carry):
            sls = [pl.ds((i * 2 + e) * 16, 16) for e in range(2)]
            dgs = [dinv_t[sl] + 1.0 for sl in sls]
            sqs = [0.5 * (dg + 1.0) for dg in dgs]
            for _ in range(14):
                sqs = [0.5 * (sq + dg / sq) for sq, dg in zip(sqs, dgs)]
            for sl, sq in zip(sls, sqs):
                dinv_t[sl] = 1.0 / sq
            return carry
        lax.fori_loop(0, NP // 32, rsq, 0)

        def sq(i, carry):
            v = dinv_t[pl.ds(t * RPT + i * 16, 16)]
            sq_buf[pl.ds(i * 16, 16)] = v * v
            return carry
        lax.fori_loop(0, RPT // 16, sq, 0)
        pltpu.sync_copy(sq_buf, selfnorm_hbm.at[pl.ds(s * NP + t * RPT, RPT)])

        sadd = s * NP

        def norm_step(u, carry):
            for j in range(SUPE // 16):
                sl = pl.ds(u * SUPE + j * 16, 16)
                osl = pl.ds(j * 16, 16)
                sv = srcb[sl]
                a = plsc.load_gather(dinv_t, [sv])
                b = plsc.load_gather(dinv_t, [dstb[sl]])
                nbuf[osl] = a * ewb[sl] * b
                abuf[osl] = sv + sadd
            off = base + u * SUPE
            pltpu.sync_copy(nbuf, norm_hbm.at[pl.ds(off, SUPE)])
            pltpu.sync_copy(abuf, srcadj_hbm.at[pl.ds(off, SUPE)])
            return carry
        lax.fori_loop(0, NSUP, norm_step, 0)
        plsc.subcore_barrier()


@jax.jit
def _sc_prep(src, dst, ew):
    return pl.kernel(
        _sc_prep_body,
        out_type=(
            jax.ShapeDtypeStruct((S * NP,), jnp.float32),
            jax.ShapeDtypeStruct((S * E,), jnp.float32),
            jax.ShapeDtypeStruct((S * E,), jnp.int32),
        ),
        mesh=_get_mesh(),
        scratch_types=[
            pltpu.VMEM((EPT,), jnp.int32),
            pltpu.VMEM((EPT,), jnp.int32),
            pltpu.VMEM((EPT,), jnp.float32),
            pltpu.VMEM((CH,), jnp.int32),
            pltpu.VMEM((CH,), jnp.int32),
            pltpu.VMEM((SUPE,), jnp.float32),
            pltpu.VMEM((SUPE,), jnp.int32),
            pltpu.VMEM((NP,), jnp.float32),
            pltpu.VMEM((RPT,), jnp.float32),
            pltpu.VMEM((RPT,), jnp.float32),
            pltpu.SemaphoreType.DMA,
            pltpu.SemaphoreType.DMA,
            pltpu.VMEM_SHARED((NP,), jnp.float32),
        ],
        compiler_params=pltpu.CompilerParams(needs_layout_passes=False),
        name="sc_prep",
    )(src, dst, ew)


def _sc_spmm_body(hw_hbm, srcadj_hbm, dst_hbm, norm_hbm, out_hbm,
                  sadj, dstg, nbuf, dsm0, dsm1, rows0, rows1,
                  zrows, semg0, semg1, sems0, sems1, acc_sh):
    c = lax.axis_index("c")
    t = lax.axis_index("s")
    rows_refs = (rows0, rows1)
    dsm_refs = (dsm0, dsm1)
    semg = (semg0, semg1)
    sems = (sems0, sems1)

    def zz(i, carry):
        for g in range(D // 16):
            zrows[i, pl.ds(g * 16, 16)] = _zero16()
        return carry
    lax.fori_loop(0, CH, zz, 0)

    for k in range(SNAP_PER_SC):
        s = c * SNAP_PER_SC + k
        for r in range(RPT // CH):
            pltpu.sync_copy(zrows, acc_sh.at[pl.ds(t * RPT + r * CH, CH)])
        plsc.subcore_barrier()

        base_e = s * E + t * EPT

        def super_body(u, carry):
            off = base_e + u * SUPE
            pltpu.sync_copy(srcadj_hbm.at[pl.ds(off, SUPE)], sadj)
            pltpu.sync_copy(norm_hbm.at[pl.ds(off, SUPE)], nbuf)
            pltpu.sync_copy(dst_hbm.at[pl.ds(off, SUPE)], dstg)
            scd = [None, None]
            gcur = pltpu.async_copy(
                hw_hbm.at[sadj.at[pl.ds(0, CH)]], rows0, semg0)
            for kk in range(SCK):
                b = kk & 1
                nb = 1 - b
                rb = rows_refs[b]
                for g in range(CH // 16):
                    dsm_refs[b][pl.ds(g * 16, 16)] = (
                        dstg[pl.ds(kk * CH + g * 16, 16)])
                gcur.wait()
                if kk < SCK - 1:
                    if scd[nb] is not None:
                        scd[nb].wait()
                    gcur = pltpu.async_copy(
                        hw_hbm.at[sadj.at[pl.ds((kk + 1) * CH, CH)]],
                        rows_refs[nb], semg[nb])
                nbase = kk * CH

                def scale(jj, carry2):
                    for e in range(2):
                        j = jj * 2 + e
                        bc = plsc.load_gather(
                            nbuf,
                            [jnp.zeros((16,), jnp.int32) + (nbase + j)])
                        vals = [rb[j, pl.ds(g * 16, 16)]
                                for g in range(D // 16)]
                        for g in range(D // 16):
                            rb[j, pl.ds(g * 16, 16)] = vals[g] * bc
                    return carry2
                lax.fori_loop(0, CH // 2, scale, 0)
                scd[b] = pltpu.async_copy(rb, acc_sh.at[dsm_refs[b]],
                                          sems[b], add=True)
            scd[0].wait()
            scd[1].wait()
            return carry
        lax.fori_loop(0, NSUP, super_body, 0)
        plsc.subcore_barrier()
        pltpu.sync_copy(acc_sh.at[pl.ds(t * RPT, RPT)],
                        out_hbm.at[pl.ds(s * NP + t * RPT, RPT)])
        plsc.subcore_barrier()


@jax.jit
def _sc_spmm(hw_flat, srcadj, dst, norm):
    return pl.kernel(
        _sc_spmm_body,
        out_type=jax.ShapeDtypeStruct((S * NP, D), jnp.float32),
        mesh=_get_mesh(),
        scratch_types=[
            pltpu.VMEM((SUPE,), jnp.int32),
            pltpu.VMEM((SUPE,), jnp.int32),
            pltpu.VMEM((SUPE,), jnp.float32),
            pltpu.VMEM((CH,), jnp.int32),
            pltpu.VMEM((CH,), jnp.int32),
            pltpu.VMEM((CH, D), jnp.float32),
            pltpu.VMEM((CH, D), jnp.float32),
            pltpu.VMEM((CH, D), jnp.float32),
            pltpu.SemaphoreType.DMA,
            pltpu.SemaphoreType.DMA,
            pltpu.SemaphoreType.DMA,
            pltpu.SemaphoreType.DMA,
            pltpu.VMEM_SHARED((NP, D), jnp.float32),
        ],
        compiler_params=pltpu.CompilerParams(needs_layout_passes=False),
        name="sc_spmm",
    )(hw_flat, srcadj, dst, norm)


def _tc_mm_body(x_ref, w_ref, o_ref):
    o_ref[...] = jnp.dot(x_ref[0], w_ref[...],
                         preferred_element_type=jnp.float32)[None]


@jax.jit
def _tc_matmul(h, w):
    return pl.pallas_call(
        _tc_mm_body,
        grid=(S, NB),
        in_specs=[
            pl.BlockSpec((1, BN, D), lambda sx, i: (sx, i, 0)),
            pl.BlockSpec((D, D), lambda sx, i: (0, 0)),
        ],
        out_specs=pl.BlockSpec((1, BN, D), lambda sx, i: (sx, i, 0)),
        out_shape=jax.ShapeDtypeStruct((S, NP, D), jnp.float32),
    )(h, w)


def _finish(acc, hw, sn, b, g, bb):
    tv = acc + sn * hw + b
    mu = jnp.mean(tv, axis=-1, keepdims=True)
    var = jnp.mean((tv - mu) ** 2, axis=-1, keepdims=True)
    hv = (tv - mu) / jnp.sqrt(var + 1e-5) * g + bb
    return jnp.maximum(hv, 0.0)


def _tc_mid_body(acc_ref, hw_ref, sn_ref, b_ref, g_ref, bb_ref, wn_ref, o_ref):
    hv = _finish(acc_ref[0], hw_ref[0], sn_ref[0], b_ref[...], g_ref[...],
                 bb_ref[...])
    o_ref[...] = jnp.dot(hv, wn_ref[...],
                         preferred_element_type=jnp.float32)[None]


def _tc_last_body(acc_ref, hw_ref, sn_ref, b_ref, g_ref, bb_ref, o_ref):
    o_ref[...] = _finish(acc_ref[0], hw_ref[0], sn_ref[0], b_ref[...],
                         g_ref[...], bb_ref[...])[None]


def _layer_specs(with_w):
    specs = [
        pl.BlockSpec((1, BN, D), lambda sx, i: (sx, i, 0)),
        pl.BlockSpec((1, BN, D), lambda sx, i: (sx, i, 0)),
        pl.BlockSpec((1, BN, 1), lambda sx, i: (sx, i, 0)),
        pl.BlockSpec((1, D), lambda sx, i: (0, 0)),
        pl.BlockSpec((1, D), lambda sx, i: (0, 0)),
        pl.BlockSpec((1, D), lambda sx, i: (0, 0)),
    ]
    if with_w:
        specs.append(pl.BlockSpec((D, D), lambda sx, i: (0, 0)))
    return specs


@jax.jit
def _tc_mid(acc, hw, sn, b, g, bb, wn):
    return pl.pallas_call(
        _tc_mid_body,
        grid=(S, NB),
        in_specs=_layer_specs(True),
        out_specs=pl.BlockSpec((1, BN, D), lambda sx, i: (sx, i, 0)),
        out_shape=jax.ShapeDtypeStruct((S, NP, D), jnp.float32),
    )(acc, hw, sn, b, g, bb, wn)


@jax.jit
def _tc_last(acc, hw, sn, b, g, bb):
    return pl.pallas_call(
        _tc_last_body,
        grid=(S, NB),
        in_specs=_layer_specs(False),
        out_specs=pl.BlockSpec((1, BN, D), lambda sx, i: (sx, i, 0)),
        out_shape=jax.ShapeDtypeStruct((S, NP, D), jnp.float32),
    )(acc, hw, sn, b, g, bb)


def _tc_gru_body(f_ref, wih_ref, whh_ref, bih_ref, bhh_ref, o_ref):
    h = jnp.zeros((BN, D), jnp.float32)
    wih = wih_ref[...]
    whh = whh_ref[...]
    for s in range(S):
        xt = f_ref[s]
        gi = jnp.dot(xt, wih, preferred_element_type=jnp.float32)
        gh = jnp.dot(h, whh, preferred_element_type=jnp.float32)
        r = jax.nn.sigmoid(gi[:, 0:D] + bih_ref[0] + gh[:, 0:D] + bhh_ref[0])
        z = jax.nn.sigmoid(gi[:, D:2 * D] + bih_ref[1]
                           + gh[:, D:2 * D] + bhh_ref[1])
        nv = jnp.tanh(gi[:, 2 * D:] + bih_ref[2]
                      + r * (gh[:, 2 * D:] + bhh_ref[2]))
        h = (1.0 - z) * nv + z * h
    o_ref[...] = h


@jax.jit
def _tc_gru(feats, wih_t, whh_t, bih, bhh):
    return pl.pallas_call(
        _tc_gru_body,
        grid=(NB,),
        in_specs=[
            pl.BlockSpec((S, BN, D), lambda i: (0, i, 0)),
            pl.BlockSpec((D, 3 * D), lambda i: (0, 0)),
            pl.BlockSpec((D, 3 * D), lambda i: (0, 0)),
            pl.BlockSpec((3, D), lambda i: (0, 0)),
            pl.BlockSpec((3, D), lambda i: (0, 0)),
        ],
        out_specs=pl.BlockSpec((BN, D), lambda i: (i, 0)),
        out_shape=jax.ShapeDtypeStruct((NP, D), jnp.float32),
    )(feats, wih_t, whh_t, bih, bhh)


def kernel(x, edge_index, edge_weight, gcn_W, gcn_b, ln_g, ln_b,
           W_ih, W_hh, b_ih, b_hh):
    src = edge_index[:, 0, :].reshape(S * E)
    dst = edge_index[:, 1, :].reshape(S * E)
    ew = edge_weight.reshape(S * E)
    xp = jnp.pad(x, ((0, 0), (0, NP - N), (0, 0)))

    selfnorm_flat, norm, srcadj = _sc_prep(src, dst, ew)
    selfnorm = selfnorm_flat.reshape(S, NP, 1)

    hw = _tc_matmul(xp, gcn_W[0])
    feats = None
    for l in range(NLAYER):
        acc = _sc_spmm(hw.reshape(S * NP, D), srcadj, dst,
                       norm).reshape(S, NP, D)
        args = (acc, hw, selfnorm, gcn_b[l][None], ln_g[l][None],
                ln_b[l][None])
        if l < NLAYER - 1:
            hw = _tc_mid(*args, gcn_W[l + 1])
        else:
            feats = _tc_last(*args)

    hstate = _tc_gru(feats, W_ih.T, W_hh.T,
                     b_ih.reshape(3, D), b_hh.reshape(3, D))
    return hstate[:N]


# speedup vs baseline: 1.9050x; 1.0857x over previous
import jax
import jax.numpy as jnp
from jax import lax
from jax.experimental import pallas as pl
from jax.experimental.pallas import tpu as pltpu
from jax.experimental.pallas import tpu_sc as plsc

N = 10000
D = 128
E = 320000
S = 4
NLAYER = 3

NP = 10240
NSC = 2
NT = 16
SNAP_PER_SC = S // NSC
EPT = E // NT
CH = 80
NCHUNK = EPT // CH
SCK = 25
SUPE = SCK * CH
NSUP = NCHUNK // SCK
RPT = NP // NT
BN = 1024
NB = NP // BN


def _get_mesh():
    return plsc.VectorSubcoreMesh(core_axis_name="c", subcore_axis_name="s",
                                  num_cores=NSC, num_subcores=NT)


def _zero16():
    return jnp.zeros((16,), jnp.float32)


def _sc_prep_body(src_hbm, dst_hbm, ew_hbm, selfnorm_hbm, norm_hbm,
                  srcadj_hbm,
                  srcb, dstb, ewb, dsmall, dsmall2, nbuf, abuf, dinv_t,
                  sq_buf, zbuf, semd0, semd1, deg_sh):
    c = lax.axis_index("c")
    t = lax.axis_index("s")

    def zz(i, carry):
        zbuf[pl.ds(i * 16, 16)] = _zero16()
        return carry
    lax.fori_loop(0, RPT // 16, zz, 0)

    for k in range(SNAP_PER_SC):
        s = c * SNAP_PER_SC + k
        base = s * E + t * EPT

        pltpu.sync_copy(src_hbm.at[pl.ds(base, EPT)], srcb)
        pltpu.sync_copy(dst_hbm.at[pl.ds(base, EPT)], dstb)
        pltpu.sync_copy(ew_hbm.at[pl.ds(base, EPT)], ewb)

        pltpu.sync_copy(zbuf, deg_sh.at[pl.ds(t * RPT, RPT)])
        plsc.subcore_barrier()

        def deg_step(i, carry):
            for e in range(2):
                kk = i * 2 + e
                dsm = dsmall if e == 0 else dsmall2
                sem = semd0 if e == 0 else semd1

                @pl.when(i >= 1)
                def _():
                    pltpu.make_async_copy(ewb.at[pl.ds(0, CH)],
                                          deg_sh.at[dsm], sem).wait()
                for g in range(CH // 16):
                    dsm[pl.ds(g * 16, 16)] = dstb[pl.ds(kk * CH + g * 16, 16)]
                pltpu.async_copy(ewb.at[pl.ds(kk * CH, CH)],
                                 deg_sh.at[dsm], sem, add=True)
            return carry
        lax.fori_loop(0, NCHUNK // 2, deg_step, 0)
        pltpu.make_async_copy(ewb.at[pl.ds(0, CH)],
                              deg_sh.at[dsmall], semd0).wait()
        pltpu.make_async_copy(ewb.at[pl.ds(0, CH)],
                              deg_sh.at[dsmall2], semd1).wait()
        plsc.subcore_barrier()

        pltpu.sync_copy(deg_sh, dinv_t)

        def rsq(i, carry):
            sls = [pl.ds((i * 2 + e) * 16, 16) for e in range(2)]
            dgs = [dinv_t[sl] + 1.0 for sl in sls]
            sqs = [0.5 * (dg + 1.0) for dg in dgs]
            for _ in range(14):
                sqs = [0.5 * (sq + dg / sq) for sq, dg in zip(sqs, dgs)]
            for sl, sq in zip(sls, sqs):
                dinv_t[sl] = 1.0 / sq
            return carry
        lax.fori_loop(0, NP // 32, rsq, 0)

        def sq(i, carry):
            v = dinv_t[pl.ds(t * RPT + i * 16, 16)]
            sq_buf[pl.ds(i * 16, 16)] = v * v
            return carry
        lax.fori_loop(0, RPT // 16, sq, 0)
        pltpu.sync_copy(sq_buf, selfnorm_hbm.at[pl.ds(s * NP + t * RPT, RPT)])

        sadd = s * NP

        def norm_step(u, carry):
            for j in range(SUPE // 16):
                sl = pl.ds(u * SUPE + j * 16, 16)
                osl = pl.ds(j * 16, 16)
                sv = srcb[sl]
                a = plsc.load_gather(dinv_t, [sv])
                b = plsc.load_gather(dinv_t, [dstb[sl]])
                nbuf[osl] = a * ewb[sl] * b
                abuf[osl] = sv + sadd
            off = base + u * SUPE
            pltpu.sync_copy(nbuf, norm_hbm.at[pl.ds(off, SUPE)])
            pltpu.sync_copy(abuf, srcadj_hbm.at[pl.ds(off, SUPE)])
            return carry
        lax.fori_loop(0, NSUP, norm_step, 0)
        plsc.subcore_barrier()


@jax.jit
def _sc_prep(src, dst, ew):
    return pl.kernel(
        _sc_prep_body,
        out_type=(
            jax.ShapeDtypeStruct((S * NP,), jnp.float32),
            jax.ShapeDtypeStruct((S * E,), jnp.float32),
            jax.ShapeDtypeStruct((S * E,), jnp.int32),
        ),
        mesh=_get_mesh(),
        scratch_types=[
            pltpu.VMEM((EPT,), jnp.int32),
            pltpu.VMEM((EPT,), jnp.int32),
            pltpu.VMEM((EPT,), jnp.float32),
            pltpu.VMEM((CH,), jnp.int32),
            pltpu.VMEM((CH,), jnp.int32),
            pltpu.VMEM((SUPE,), jnp.float32),
            pltpu.VMEM((SUPE,), jnp.int32),
            pltpu.VMEM((NP,), jnp.float32),
            pltpu.VMEM((RPT,), jnp.float32),
            pltpu.VMEM((RPT,), jnp.float32),
            pltpu.SemaphoreType.DMA,
            pltpu.SemaphoreType.DMA,
            pltpu.VMEM_SHARED((NP,), jnp.float32),
        ],
        compiler_params=pltpu.CompilerParams(needs_layout_passes=False),
        name="sc_prep",
    )(src, dst, ew)


def _sc_spmm_body(hw_hbm, srcadj_hbm, dst_hbm, norm_hbm, out_hbm,
                  sadj, dstg, nbuf, dsm0, dsm1, rows0, rows1,
                  zrows, semg0, semg1, sems0, sems1, acc_sh):
    c = lax.axis_index("c")
    t = lax.axis_index("s")
    rows_refs = (rows0, rows1)
    dsm_refs = (dsm0, dsm1)
    semg = (semg0, semg1)
    sems = (sems0, sems1)

    def zz(i, carry):
        for g in range(D // 16):
            zrows[i, pl.ds(g * 16, 16)] = _zero16()
        return carry
    lax.fori_loop(0, CH, zz, 0)

    for k in range(SNAP_PER_SC):
        s = c * SNAP_PER_SC + k
        for r in range(RPT // CH):
            pltpu.sync_copy(zrows, acc_sh.at[pl.ds(t * RPT + r * CH, CH)])
        plsc.subcore_barrier()

        base_e = s * E + t * EPT

        def super_body(u, carry):
            off = base_e + u * SUPE
            pltpu.sync_copy(srcadj_hbm.at[pl.ds(off, SUPE)], sadj)
            pltpu.sync_copy(norm_hbm.at[pl.ds(off, SUPE)], nbuf)
            pltpu.sync_copy(dst_hbm.at[pl.ds(off, SUPE)], dstg)
            scd = [None, None]
            gcur = pltpu.async_copy(
                hw_hbm.at[sadj.at[pl.ds(0, CH)]], rows0, semg0)
            for kk in range(SCK):
                b = kk & 1
                nb = 1 - b
                rb = rows_refs[b]
                for g in range(CH // 16):
                    dsm_refs[b][pl.ds(g * 16, 16)] = (
                        dstg[pl.ds(kk * CH + g * 16, 16)])
                gcur.wait()
                if kk < SCK - 1:
                    if scd[nb] is not None:
                        scd[nb].wait()
                    gcur = pltpu.async_copy(
                        hw_hbm.at[sadj.at[pl.ds((kk + 1) * CH, CH)]],
                        rows_refs[nb], semg[nb])
                nbase = kk * CH

                def scale(jj, carry2):
                    for e in range(2):
                        j = jj * 2 + e
                        bc = plsc.load_gather(
                            nbuf,
                            [jnp.zeros((16,), jnp.int32) + (nbase + j)])
                        vals = [rb[j, pl.ds(g * 16, 16)]
                                for g in range(D // 16)]
                        for g in range(D // 16):
                            rb[j, pl.ds(g * 16, 16)] = vals[g] * bc
                    return carry2
                lax.fori_loop(0, CH // 2, scale, 0)
                scd[b] = pltpu.async_copy(rb, acc_sh.at[dsm_refs[b]],
                                          sems[b], add=True)
            scd[0].wait()
            scd[1].wait()
            return carry
        lax.fori_loop(0, NSUP, super_body, 0)
        plsc.subcore_barrier()
        pltpu.sync_copy(acc_sh.at[pl.ds(t * RPT, RPT)],
                        out_hbm.at[pl.ds(s * NP + t * RPT, RPT)])
        plsc.subcore_barrier()


@jax.jit
def _sc_spmm(hw_flat, srcadj, dst, norm):
    return pl.kernel(
        _sc_spmm_body,
        out_type=jax.ShapeDtypeStruct((S * NP, D), jnp.float32),
        mesh=_get_mesh(),
        scratch_types=[
            pltpu.VMEM((SUPE,), jnp.int32),
            pltpu.VMEM((SUPE,), jnp.int32),
            pltpu.VMEM((SUPE,), jnp.float32),
            pltpu.VMEM((CH,), jnp.int32),
            pltpu.VMEM((CH,), jnp.int32),
            pltpu.VMEM((CH, D), jnp.float32),
            pltpu.VMEM((CH, D), jnp.float32),
            pltpu.VMEM((CH, D), jnp.float32),
            pltpu.SemaphoreType.DMA,
            pltpu.SemaphoreType.DMA,
            pltpu.SemaphoreType.DMA,
            pltpu.SemaphoreType.DMA,
            pltpu.VMEM_SHARED((NP, D), jnp.float32),
        ],
        compiler_params=pltpu.CompilerParams(needs_layout_passes=False),
        name="sc_spmm",
    )(hw_flat, srcadj, dst, norm)


def _tc_mm_body(x_ref, w_ref, o_ref):
    o_ref[...] = jnp.dot(x_ref[0], w_ref[...],
                         preferred_element_type=jnp.float32)[None]


@jax.jit
def _tc_matmul(h, w):
    return pl.pallas_call(
        _tc_mm_body,
        grid=(S, NB),
        in_specs=[
            pl.BlockSpec((1, BN, D), lambda sx, i: (sx, i, 0)),
            pl.BlockSpec((D, D), lambda sx, i: (0, 0)),
        ],
        out_specs=pl.BlockSpec((1, BN, D), lambda sx, i: (sx, i, 0)),
        out_shape=jax.ShapeDtypeStruct((S, NP, D), jnp.float32),
    )(h, w)


def _finish(acc, hw, sn, b, g, bb):
    tv = acc + sn * hw + b
    mu = jnp.mean(tv, axis=-1, keepdims=True)
    var = jnp.mean((tv - mu) ** 2, axis=-1, keepdims=True)
    hv = (tv - mu) / jnp.sqrt(var + 1e-5) * g + bb
    return jnp.maximum(hv, 0.0)


def _tc_mid_body(acc_ref, hw_ref, sn_ref, b_ref, g_ref, bb_ref, wn_ref, o_ref):
    hv = _finish(acc_ref[0], hw_ref[0], sn_ref[0], b_ref[...], g_ref[...],
                 bb_ref[...])
    o_ref[...] = jnp.dot(hv, wn_ref[...],
                         preferred_element_type=jnp.float32)[None]


def _tc_last_body(acc_ref, hw_ref, sn_ref, b_ref, g_ref, bb_ref, o_ref):
    o_ref[...] = _finish(acc_ref[0], hw_ref[0], sn_ref[0], b_ref[...],
                         g_ref[...], bb_ref[...])[None]


def _layer_specs(with_w):
    specs = [
        pl.BlockSpec((1, BN, D), lambda sx, i: (sx, i, 0)),
        pl.BlockSpec((1, BN, D), lambda sx, i: (sx, i, 0)),
        pl.BlockSpec((1, BN, 1), lambda sx, i: (sx, i, 0)),
        pl.BlockSpec((1, D), lambda sx, i: (0, 0)),
        pl.BlockSpec((1, D), lambda sx, i: (0, 0)),
        pl.BlockSpec((1, D), lambda sx, i: (0, 0)),
    ]
    if with_w:
        specs.append(pl.BlockSpec((D, D), lambda sx, i: (0, 0)))
    return specs


@jax.jit
def _tc_mid(acc, hw, sn, b, g, bb, wn):
    return pl.pallas_call(
        _tc_mid_body,
        grid=(S, NB),
        in_specs=_layer_specs(True),
        out_specs=pl.BlockSpec((1, BN, D), lambda sx, i: (sx, i, 0)),
        out_shape=jax.ShapeDtypeStruct((S, NP, D), jnp.float32),
    )(acc, hw, sn, b, g, bb, wn)


@jax.jit
def _tc_last(acc, hw, sn, b, g, bb):
    return pl.pallas_call(
        _tc_last_body,
        grid=(S, NB),
        in_specs=_layer_specs(False),
        out_specs=pl.BlockSpec((1, BN, D), lambda sx, i: (sx, i, 0)),
        out_shape=jax.ShapeDtypeStruct((S, NP, D), jnp.float32),
    )(acc, hw, sn, b, g, bb)


def _tc_gru_body(f_ref, wih_ref, whh_ref, bih_ref, bhh_ref, o_ref):
    h = jnp.zeros((BN, D), jnp.float32)
    wih = wih_ref[...]
    whh = whh_ref[...]
    for s in range(S):
        xt = f_ref[s]
        gi = jnp.dot(xt, wih, preferred_element_type=jnp.float32)
        gh = jnp.dot(h, whh, preferred_element_type=jnp.float32)
        r = jax.nn.sigmoid(gi[:, 0:D] + bih_ref[0] + gh[:, 0:D] + bhh_ref[0])
        z = jax.nn.sigmoid(gi[:, D:2 * D] + bih_ref[1]
                           + gh[:, D:2 * D] + bhh_ref[1])
        nv = jnp.tanh(gi[:, 2 * D:] + bih_ref[2]
                      + r * (gh[:, 2 * D:] + bhh_ref[2]))
        h = (1.0 - z) * nv + z * h
    o_ref[...] = h


@jax.jit
def _tc_gru(feats, wih_t, whh_t, bih, bhh):
    return pl.pallas_call(
        _tc_gru_body,
        grid=(NB,),
        in_specs=[
            pl.BlockSpec((S, BN, D), lambda i: (0, i, 0)),
            pl.BlockSpec((D, 3 * D), lambda i: (0, 0)),
            pl.BlockSpec((D, 3 * D), lambda i: (0, 0)),
            pl.BlockSpec((3, D), lambda i: (0, 0)),
            pl.BlockSpec((3, D), lambda i: (0, 0)),
        ],
        out_specs=pl.BlockSpec((BN, D), lambda i: (i, 0)),
        out_shape=jax.ShapeDtypeStruct((NP, D), jnp.float32),
    )(feats, wih_t, whh_t, bih, bhh)


def kernel(x, edge_index, edge_weight, gcn_W, gcn_b, ln_g, ln_b,
           W_ih, W_hh, b_ih, b_hh):
    src = edge_index[:, 0, :].reshape(S * E)
    dst = edge_index[:, 1, :].reshape(S * E)
    ew = edge_weight.reshape(S * E)
    xp = jnp.pad(x, ((0, 0), (0, NP - N), (0, 0)))

    selfnorm_flat, norm, srcadj = _sc_prep(src, dst, ew)
    selfnorm = selfnorm_flat.reshape(S, NP, 1)

    hw = _tc_matmul(xp, gcn_W[0])
    feats = None
    for l in range(NLAYER):
        acc = _sc_spmm(hw.reshape(S * NP, D), srcadj, dst,
                       norm).reshape(S, NP, D)
        args = (acc, hw, selfnorm, gcn_b[l][None], ln_g[l][None],
                ln_b[l][None])
        if l < NLAYER - 1:
            hw = _tc_mid(*args, gcn_W[l + 1])
        else:
            feats = _tc_last(*args)

    hstate = _tc_gru(feats, W_ih.T, W_hh.T,
                     b_ih.reshape(3, D), b_hh.reshape(3, D))
    return hstate[:N]


# speedup vs baseline: 1.9530x; 1.0252x over previous
import jax
import jax.numpy as jnp
from jax import lax
from jax.experimental import pallas as pl
from jax.experimental.pallas import tpu as pltpu
from jax.experimental.pallas import tpu_sc as plsc

N = 10000
D = 128
E = 320000
S = 4
NLAYER = 3

NP = 10240
NSC = 2
NT = 16
SNAP_PER_SC = S // NSC
EPT = E // NT
CH = 80
NCHUNK = EPT // CH
SCK = 50
SUPE = SCK * CH
NSUP = NCHUNK // SCK
RPT = NP // NT
BN = 1024
NB = NP // BN


def _get_mesh():
    return plsc.VectorSubcoreMesh(core_axis_name="c", subcore_axis_name="s",
                                  num_cores=NSC, num_subcores=NT)


def _zero16():
    return jnp.zeros((16,), jnp.float32)


def _sc_prep_body(src_hbm, dst_hbm, ew_hbm, selfnorm_hbm, norm_hbm,
                  srcadj_hbm,
                  srcb, dstb, ewb, dsmall, dsmall2, nbuf, abuf, dinv_t,
                  sq_buf, zbuf, semd0, semd1, deg_sh):
    c = lax.axis_index("c")
    t = lax.axis_index("s")

    def zz(i, carry):
        zbuf[pl.ds(i * 16, 16)] = _zero16()
        return carry
    lax.fori_loop(0, RPT // 16, zz, 0)

    for k in range(SNAP_PER_SC):
        s = c * SNAP_PER_SC + k
        base = s * E + t * EPT

        pltpu.sync_copy(src_hbm.at[pl.ds(base, EPT)], srcb)
        pltpu.sync_copy(dst_hbm.at[pl.ds(base, EPT)], dstb)
        pltpu.sync_copy(ew_hbm.at[pl.ds(base, EPT)], ewb)

        pltpu.sync_copy(zbuf, deg_sh.at[pl.ds(t * RPT, RPT)])
        plsc.subcore_barrier()

        def deg_step(i, carry):
            for e in range(2):
                kk = i * 2 + e
                dsm = dsmall if e == 0 else dsmall2
                sem = semd0 if e == 0 else semd1

                @pl.when(i >= 1)
                def _():
                    pltpu.make_async_copy(ewb.at[pl.ds(0, CH)],
                                          deg_sh.at[dsm], sem).wait()
                for g in range(CH // 16):
                    dsm[pl.ds(g * 16, 16)] = dstb[pl.ds(kk * CH + g * 16, 16)]
                pltpu.async_copy(ewb.at[pl.ds(kk * CH, CH)],
                                 deg_sh.at[dsm], sem, add=True)
            return carry
        lax.fori_loop(0, NCHUNK // 2, deg_step, 0)
        pltpu.make_async_copy(ewb.at[pl.ds(0, CH)],
                              deg_sh.at[dsmall], semd0).wait()
        pltpu.make_async_copy(ewb.at[pl.ds(0, CH)],
                              deg_sh.at[dsmall2], semd1).wait()
        plsc.subcore_barrier()

        pltpu.sync_copy(deg_sh, dinv_t)

        def rsq(i, carry):
            sls = [pl.ds((i * 2 + e) * 16, 16) for e in range(2)]
            dgs = [dinv_t[sl] + 1.0 for sl in sls]
            sqs = [0.5 * (dg + 1.0) for dg in dgs]
            for _ in range(14):
                sqs = [0.5 * (sq + dg / sq) for sq, dg in zip(sqs, dgs)]
            for sl, sq in zip(sls, sqs):
                dinv_t[sl] = 1.0 / sq
            return carry
        lax.fori_loop(0, NP // 32, rsq, 0)

        def sq(i, carry):
            v = dinv_t[pl.ds(t * RPT + i * 16, 16)]
            sq_buf[pl.ds(i * 16, 16)] = v * v
            return carry
        lax.fori_loop(0, RPT // 16, sq, 0)
        pltpu.sync_copy(sq_buf, selfnorm_hbm.at[pl.ds(s * NP + t * RPT, RPT)])

        sadd = s * NP

        def norm_step(u, carry):
            for j in range(SUPE // 16):
                sl = pl.ds(u * SUPE + j * 16, 16)
                osl = pl.ds(j * 16, 16)
                sv = srcb[sl]
                a = plsc.load_gather(dinv_t, [sv])
                b = plsc.load_gather(dinv_t, [dstb[sl]])
                nbuf[osl] = a * ewb[sl] * b
                abuf[osl] = sv + sadd
            off = base + u * SUPE
            pltpu.sync_copy(nbuf, norm_hbm.at[pl.ds(off, SUPE)])
            pltpu.sync_copy(abuf, srcadj_hbm.at[pl.ds(off, SUPE)])
            return carry
        lax.fori_loop(0, NSUP, norm_step, 0)
        plsc.subcore_barrier()


@jax.jit
def _sc_prep(src, dst, ew):
    return pl.kernel(
        _sc_prep_body,
        out_type=(
            jax.ShapeDtypeStruct((S * NP,), jnp.float32),
            jax.ShapeDtypeStruct((S * E,), jnp.float32),
            jax.ShapeDtypeStruct((S * E,), jnp.int32),
        ),
        mesh=_get_mesh(),
        scratch_types=[
            pltpu.VMEM((EPT,), jnp.int32),
            pltpu.VMEM((EPT,), jnp.int32),
            pltpu.VMEM((EPT,), jnp.float32),
            pltpu.VMEM((CH,), jnp.int32),
            pltpu.VMEM((CH,), jnp.int32),
            pltpu.VMEM((SUPE,), jnp.float32),
            pltpu.VMEM((SUPE,), jnp.int32),
            pltpu.VMEM((NP,), jnp.float32),
            pltpu.VMEM((RPT,), jnp.float32),
            pltpu.VMEM((RPT,), jnp.float32),
            pltpu.SemaphoreType.DMA,
            pltpu.SemaphoreType.DMA,
            pltpu.VMEM_SHARED((NP,), jnp.float32),
        ],
        compiler_params=pltpu.CompilerParams(needs_layout_passes=False),
        name="sc_prep",
    )(src, dst, ew)


def _sc_spmm_body(hw_hbm, srcadj_hbm, dst_hbm, norm_hbm, out_hbm,
                  sadj, dstg, nbuf, dsm0, dsm1, rows0, rows1,
                  zrows, semg0, semg1, sems0, sems1, acc_sh):
    c = lax.axis_index("c")
    t = lax.axis_index("s")
    rows_refs = (rows0, rows1)
    dsm_refs = (dsm0, dsm1)
    semg = (semg0, semg1)
    sems = (sems0, sems1)

    def zz(i, carry):
        for g in range(D // 16):
            zrows[i, pl.ds(g * 16, 16)] = _zero16()
        return carry
    lax.fori_loop(0, CH, zz, 0)

    for k in range(SNAP_PER_SC):
        s = c * SNAP_PER_SC + k
        for r in range(RPT // CH):
            pltpu.sync_copy(zrows, acc_sh.at[pl.ds(t * RPT + r * CH, CH)])
        plsc.subcore_barrier()

        base_e = s * E + t * EPT

        def super_body(u, carry):
            off = base_e + u * SUPE
            pltpu.sync_copy(srcadj_hbm.at[pl.ds(off, SUPE)], sadj)
            pltpu.sync_copy(norm_hbm.at[pl.ds(off, SUPE)], nbuf)
            pltpu.sync_copy(dst_hbm.at[pl.ds(off, SUPE)], dstg)
            scd = [None, None]
            gcur = pltpu.async_copy(
                hw_hbm.at[sadj.at[pl.ds(0, CH)]], rows0, semg0)
            for kk in range(SCK):
                b = kk & 1
                nb = 1 - b
                rb = rows_refs[b]
                for g in range(CH // 16):
                    dsm_refs[b][pl.ds(g * 16, 16)] = (
                        dstg[pl.ds(kk * CH + g * 16, 16)])
                gcur.wait()
                if kk < SCK - 1:
                    if scd[nb] is not None:
                        scd[nb].wait()
                    gcur = pltpu.async_copy(
                        hw_hbm.at[sadj.at[pl.ds((kk + 1) * CH, CH)]],
                        rows_refs[nb], semg[nb])
                nbase = kk * CH

                def scale(jj, carry2):
                    for e in range(2):
                        j = jj * 2 + e
                        bc = plsc.load_gather(
                            nbuf,
                            [jnp.zeros((16,), jnp.int32) + (nbase + j)])
                        vals = [rb[j, pl.ds(g * 16, 16)]
                                for g in range(D // 16)]
                        for g in range(D // 16):
                            rb[j, pl.ds(g * 16, 16)] = vals[g] * bc
                    return carry2
                lax.fori_loop(0, CH // 2, scale, 0)
                scd[b] = pltpu.async_copy(rb, acc_sh.at[dsm_refs[b]],
                                          sems[b], add=True)
            scd[0].wait()
            scd[1].wait()
            return carry
        lax.fori_loop(0, NSUP, super_body, 0)
        plsc.subcore_barrier()
        pltpu.sync_copy(acc_sh.at[pl.ds(t * RPT, RPT)],
                        out_hbm.at[pl.ds(s * NP + t * RPT, RPT)])
        plsc.subcore_barrier()


@jax.jit
def _sc_spmm(hw_flat, srcadj, dst, norm):
    return pl.kernel(
        _sc_spmm_body,
        out_type=jax.ShapeDtypeStruct((S * NP, D), jnp.float32),
        mesh=_get_mesh(),
        scratch_types=[
            pltpu.VMEM((SUPE,), jnp.int32),
            pltpu.VMEM((SUPE,), jnp.int32),
            pltpu.VMEM((SUPE,), jnp.float32),
            pltpu.VMEM((CH,), jnp.int32),
            pltpu.VMEM((CH,), jnp.int32),
            pltpu.VMEM((CH, D), jnp.float32),
            pltpu.VMEM((CH, D), jnp.float32),
            pltpu.VMEM((CH, D), jnp.float32),
            pltpu.SemaphoreType.DMA,
            pltpu.SemaphoreType.DMA,
            pltpu.SemaphoreType.DMA,
            pltpu.SemaphoreType.DMA,
            pltpu.VMEM_SHARED((NP, D), jnp.float32),
        ],
        compiler_params=pltpu.CompilerParams(needs_layout_passes=False),
        name="sc_spmm",
    )(hw_flat, srcadj, dst, norm)


def _tc_mm_body(x_ref, w_ref, o_ref):
    o_ref[...] = jnp.dot(x_ref[0], w_ref[...],
                         preferred_element_type=jnp.float32)[None]


@jax.jit
def _tc_matmul(h, w):
    return pl.pallas_call(
        _tc_mm_body,
        grid=(S, NB),
        in_specs=[
            pl.BlockSpec((1, BN, D), lambda sx, i: (sx, i, 0)),
            pl.BlockSpec((D, D), lambda sx, i: (0, 0)),
        ],
        out_specs=pl.BlockSpec((1, BN, D), lambda sx, i: (sx, i, 0)),
        out_shape=jax.ShapeDtypeStruct((S, NP, D), jnp.float32),
    )(h, w)


def _finish(acc, hw, sn, b, g, bb):
    tv = acc + sn * hw + b
    mu = jnp.mean(tv, axis=-1, keepdims=True)
    var = jnp.mean((tv - mu) ** 2, axis=-1, keepdims=True)
    hv = (tv - mu) / jnp.sqrt(var + 1e-5) * g + bb
    return jnp.maximum(hv, 0.0)


def _tc_mid_body(acc_ref, hw_ref, sn_ref, b_ref, g_ref, bb_ref, wn_ref, o_ref):
    hv = _finish(acc_ref[0], hw_ref[0], sn_ref[0], b_ref[...], g_ref[...],
                 bb_ref[...])
    o_ref[...] = jnp.dot(hv, wn_ref[...],
                         preferred_element_type=jnp.float32)[None]


def _tc_last_body(acc_ref, hw_ref, sn_ref, b_ref, g_ref, bb_ref, o_ref):
    o_ref[...] = _finish(acc_ref[0], hw_ref[0], sn_ref[0], b_ref[...],
                         g_ref[...], bb_ref[...])[None]


def _layer_specs(with_w):
    specs = [
        pl.BlockSpec((1, BN, D), lambda sx, i: (sx, i, 0)),
        pl.BlockSpec((1, BN, D), lambda sx, i: (sx, i, 0)),
        pl.BlockSpec((1, BN, 1), lambda sx, i: (sx, i, 0)),
        pl.BlockSpec((1, D), lambda sx, i: (0, 0)),
        pl.BlockSpec((1, D), lambda sx, i: (0, 0)),
        pl.BlockSpec((1, D), lambda sx, i: (0, 0)),
    ]
    if with_w:
        specs.append(pl.BlockSpec((D, D), lambda sx, i: (0, 0)))
    return specs


@jax.jit
def _tc_mid(acc, hw, sn, b, g, bb, wn):
    return pl.pallas_call(
        _tc_mid_body,
        grid=(S, NB),
        in_specs=_layer_specs(True),
        out_specs=pl.BlockSpec((1, BN, D), lambda sx, i: (sx, i, 0)),
        out_shape=jax.ShapeDtypeStruct((S, NP, D), jnp.float32),
    )(acc, hw, sn, b, g, bb, wn)


@jax.jit
def _tc_last(acc, hw, sn, b, g, bb):
    return pl.pallas_call(
        _tc_last_body,
        grid=(S, NB),
        in_specs=_layer_specs(False),
        out_specs=pl.BlockSpec((1, BN, D), lambda sx, i: (sx, i, 0)),
        out_shape=jax.ShapeDtypeStruct((S, NP, D), jnp.float32),
    )(acc, hw, sn, b, g, bb)


def _tc_gru_body(f_ref, wih_ref, whh_ref, bih_ref, bhh_ref, o_ref):
    h = jnp.zeros((BN, D), jnp.float32)
    wih = wih_ref[...]
    whh = whh_ref[...]
    for s in range(S):
        xt = f_ref[s]
        gi = jnp.dot(xt, wih, preferred_element_type=jnp.float32)
        gh = jnp.dot(h, whh, preferred_element_type=jnp.float32)
        r = jax.nn.sigmoid(gi[:, 0:D] + bih_ref[0] + gh[:, 0:D] + bhh_ref[0])
        z = jax.nn.sigmoid(gi[:, D:2 * D] + bih_ref[1]
                           + gh[:, D:2 * D] + bhh_ref[1])
        nv = jnp.tanh(gi[:, 2 * D:] + bih_ref[2]
                      + r * (gh[:, 2 * D:] + bhh_ref[2]))
        h = (1.0 - z) * nv + z * h
    o_ref[...] = h


@jax.jit
def _tc_gru(feats, wih_t, whh_t, bih, bhh):
    return pl.pallas_call(
        _tc_gru_body,
        grid=(NB,),
        in_specs=[
            pl.BlockSpec((S, BN, D), lambda i: (0, i, 0)),
            pl.BlockSpec((D, 3 * D), lambda i: (0, 0)),
            pl.BlockSpec((D, 3 * D), lambda i: (0, 0)),
            pl.BlockSpec((3, D), lambda i: (0, 0)),
            pl.BlockSpec((3, D), lambda i: (0, 0)),
        ],
        out_specs=pl.BlockSpec((BN, D), lambda i: (i, 0)),
        out_shape=jax.ShapeDtypeStruct((NP, D), jnp.float32),
    )(feats, wih_t, whh_t, bih, bhh)


def kernel(x, edge_index, edge_weight, gcn_W, gcn_b, ln_g, ln_b,
           W_ih, W_hh, b_ih, b_hh):
    src = edge_index[:, 0, :].reshape(S * E)
    dst = edge_index[:, 1, :].reshape(S * E)
    ew = edge_weight.reshape(S * E)
    xp = jnp.pad(x, ((0, 0), (0, NP - N), (0, 0)))

    selfnorm_flat, norm, srcadj = _sc_prep(src, dst, ew)
    selfnorm = selfnorm_flat.reshape(S, NP, 1)

    hw = _tc_matmul(xp, gcn_W[0])
    feats = None
    for l in range(NLAYER):
        acc = _sc_spmm(hw.reshape(S * NP, D), srcadj, dst,
                       norm).reshape(S, NP, D)
        args = (acc, hw, selfnorm, gcn_b[l][None], ln_g[l][None],
                ln_b[l][None])
        if l < NLAYER - 1:
            hw = _tc_mid(*args, gcn_W[l + 1])
        else:
            feats = _tc_last(*args)

    hstate = _tc_gru(feats, W_ih.T, W_hh.T,
                     b_ih.reshape(3, D), b_hh.reshape(3, D))
    return hstate[:N]
